# baseline (speedup 1.0000x reference)
import functools

import jax
import jax.numpy as jnp
from jax import lax
from jax.experimental import pallas as pl
from jax.experimental.pallas import tpu as pltpu
from jax.experimental.pallas import tpu_sc as plsc

N_NODES = 10000
D_IN = 128
D_HID = 128
D_OUT = 64
N_EDGES = 160000

NC = 2
NS = 16
LANES = 128

N_PAD = 10240
ROWS_PT = N_PAD // NS
EROWS = N_EDGES // LANES
EROWS_PT = 80
EROWS_T15 = EROWS - 15 * EROWS_PT
ICHUNK = 40
ZROWS = 32
HBATCH = 10
BLK = 2000

_sc_mesh = functools.partial(
    plsc.VectorSubcoreMesh, core_axis_name="c", subcore_axis_name="s")


@functools.partial(
    pl.kernel,
    out_type=[jax.ShapeDtypeStruct((N_PAD,), jnp.float32) for _ in range(4)],
    mesh=_sc_mesh(),
    scratch_types=[
        pltpu.VMEM((EROWS_PT, LANES), jnp.int32),
        pltpu.VMEM((EROWS_PT, LANES), jnp.int32),
        pltpu.VMEM((LANES,), jnp.float32),
        pltpu.VMEM((ROWS_PT,), jnp.float32),
        pltpu.VMEM_SHARED((N_PAD,), jnp.float32),
        pltpu.VMEM_SHARED((N_PAD,), jnp.float32),
        pltpu.SemaphoreType.DMA,
    ],
    compiler_params=pltpu.CompilerParams(use_tc_tiling_on_sc=False),
)
def _hist_k(src1, dst1, src2, dst2, d0, d1, d2, d3,
            sidx, didx, ones_v, zbuf, acc_a, acc_b, sem):
    cid = lax.axis_index("c")
    tid = lax.axis_index("s")
    for i in range(LANES // 16):
        ones_v[pl.ds(i * 16, 16)] = jnp.ones((16,), jnp.float32)

    def zb(i, c):
        zbuf[pl.ds(i * 16, 16)] = jnp.zeros((16,), jnp.float32)
        return c

    lax.fori_loop(0, ROWS_PT // 16, zb, 0)
    sl = pl.ds(tid * ROWS_PT, ROWS_PT)
    pltpu.async_copy(zbuf, acc_a.at[sl], sem)
    pltpu.async_copy(zbuf, acc_b.at[sl], sem)
    pltpu.make_async_copy(zbuf, acc_a.at[sl], sem).wait()
    pltpu.make_async_copy(zbuf, acc_b.at[sl], sem).wait()
    plsc.subcore_barrier()

    def run(src_hbm, dst_hbm):
        def stage(nrows):
            esl = pl.ds(tid * EROWS_PT, nrows)
            vsl = pl.ds(0, nrows)
            pltpu.sync_copy(src_hbm.at[esl], sidx.at[vsl])
            pltpu.sync_copy(dst_hbm.at[esl], didx.at[vsl])

        pl.when(tid < NS - 1)(lambda: stage(EROWS_PT))
        pl.when(tid == NS - 1)(lambda: stage(EROWS_T15))
        nbatch = jnp.where(tid == NS - 1, EROWS_T15 // HBATCH,
                           EROWS_PT // HBATCH)

        def w(i, c):
            r = i * HBATCH
            for j in range(HBATCH):
                pltpu.async_copy(ones_v, acc_a.at[sidx.at[r + j]], sem,
                                 add=True)
                pltpu.async_copy(ones_v, acc_b.at[didx.at[r + j]], sem,
                                 add=True)
            for j in range(HBATCH):
                pltpu.make_async_copy(ones_v, acc_a.at[sidx.at[r + j]],
                                      sem).wait()
                pltpu.make_async_copy(ones_v, acc_b.at[didx.at[r + j]],
                                      sem).wait()
            return c

        lax.fori_loop(0, nbatch, w, 0)

    pl.when(cid == 0)(lambda: run(src1, dst1))
    pl.when(cid == 1)(lambda: run(src2, dst2))
    plsc.subcore_barrier()

    def out0():
        pltpu.sync_copy(acc_a.at[sl], d0.at[sl])
        pltpu.sync_copy(acc_b.at[sl], d1.at[sl])

    def out1():
        pltpu.sync_copy(acc_a.at[sl], d2.at[sl])
        pltpu.sync_copy(acc_b.at[sl], d3.at[sl])

    pl.when(cid == 0)(out0)
    pl.when(cid == 1)(out1)


def _make_agg(d):
    deep = d == D_OUT
    nbuf = 4 if deep else 2
    irows = EROWS_PT if deep else ICHUNK
    scratch = [
        pltpu.VMEM((irows, LANES), jnp.int32),
        pltpu.VMEM((irows, LANES), jnp.int32),
    ]
    scratch += [pltpu.VMEM((LANES, d), jnp.float32)] * nbuf
    scratch += [pltpu.VMEM((ZROWS, d), jnp.float32)]
    scratch += [pltpu.VMEM_SHARED((N_PAD, d), jnp.float32)]
    scratch += [pltpu.SemaphoreType.DMA] * (2 * nbuf if deep else nbuf)

    @functools.partial(
        pl.kernel,
        out_type=[jax.ShapeDtypeStruct((N_PAD, d), jnp.float32),
                  jax.ShapeDtypeStruct((N_PAD, d), jnp.float32)],
        mesh=_sc_mesh(),
        scratch_types=scratch,
        compiler_params=pltpu.CompilerParams(use_tc_tiling_on_sc=False),
    )
    def agg_k(h_a, h_b, src1, dst1, src2, dst2, out_a, out_b,
              sidx, didx, *rest):
        bufs = rest[:nbuf]
        zbuf = rest[nbuf]
        acc = rest[nbuf + 1]
        sems = rest[nbuf + 2:]
        cid = lax.axis_index("c")
        tid = lax.axis_index("s")
        sl = pl.ds(tid * ROWS_PT, ROWS_PT)

        def zb(r, c):
            for j in range(d // 16):
                zbuf[r, pl.ds(j * 16, 16)] = jnp.zeros((16,), jnp.float32)
            return c

        lax.fori_loop(0, ZROWS, zb, 0)

        def zcp(j, c):
            pltpu.async_copy(
                zbuf, acc.at[pl.ds(tid * ROWS_PT + j * ZROWS, ZROWS)],
                sems[0])
            return c

        lax.fori_loop(0, ROWS_PT // ZROWS, zcp, 0)

        def zwait(j, c):
            pltpu.make_async_copy(
                zbuf, acc.at[pl.ds(tid * ROWS_PT + j * ZROWS, ZROWS)],
                sems[0]).wait()
            return c

        lax.fori_loop(0, ROWS_PT // ZROWS, zwait, 0)
        plsc.subcore_barrier()

        def run2(h_hbm, src_hbm, dst_hbm):
            buf0, buf1 = bufs
            sem0, sem1 = sems

            def chunk(off, sz):
                base = tid * EROWS_PT + off
                csl = pl.ds(0, sz)
                pltpu.sync_copy(src_hbm.at[pl.ds(base, sz)], sidx.at[csl])
                pltpu.sync_copy(dst_hbm.at[pl.ds(base, sz)], didx.at[csl])
                pltpu.async_copy(h_hbm.at[sidx.at[0]], buf0, sem0)

                def w(i, cc):
                    r = 2 * i
                    pltpu.async_copy(h_hbm.at[sidx.at[r + 1]], buf1, sem1)
                    pltpu.make_async_copy(
                        h_hbm.at[sidx.at[r]], buf0, sem0).wait()
                    pltpu.sync_copy(buf0, acc.at[didx.at[r]], add=True)

                    @pl.when(r + 2 < sz)
                    def _():
                        pltpu.async_copy(h_hbm.at[sidx.at[r + 2]], buf0, sem0)

                    pltpu.make_async_copy(
                        h_hbm.at[sidx.at[r + 1]], buf1, sem1).wait()
                    pltpu.sync_copy(buf1, acc.at[didx.at[r + 1]], add=True)
                    return cc

                lax.fori_loop(0, sz // 2, w, 0)

            def chunks(sizes):
                off = 0
                for sz in sizes:
                    chunk(off, sz)
                    off += sz

            pl.when(tid < NS - 1)(lambda: chunks([ICHUNK, ICHUNK]))
            pl.when(tid == NS - 1)(
                lambda: chunks([ICHUNK, EROWS_T15 - ICHUNK]))

        def run4(h_hbm, src_hbm, dst_hbm):
            gsems = sems[:nbuf]
            ssems = sems[nbuf:]

            def stage(nrows):
                esl = pl.ds(tid * EROWS_PT, nrows)
                vsl = pl.ds(0, nrows)
                pltpu.sync_copy(src_hbm.at[esl], sidx.at[vsl])
                pltpu.sync_copy(dst_hbm.at[esl], didx.at[vsl])

            pl.when(tid < NS - 1)(lambda: stage(EROWS_PT))
            pl.when(tid == NS - 1)(lambda: stage(EROWS_T15))
            nrows = jnp.where(tid == NS - 1, EROWS_T15, EROWS_PT)
            pltpu.async_copy(h_hbm.at[sidx.at[0]], bufs[0], gsems[0])
            pltpu.async_copy(h_hbm.at[sidx.at[1]], bufs[1], gsems[1])

            def w(i, cc):
                for j in range(nbuf):
                    r = nbuf * i + j
                    j2 = (j + 2) % nbuf

                    @pl.when(r + 2 < nrows)
                    def _():
                        @pl.when(r >= 2)
                        def _():
                            pltpu.make_async_copy(
                                bufs[j2], acc.at[didx.at[r - 2]],
                                ssems[j2]).wait()

                        pltpu.async_copy(h_hbm.at[sidx.at[r + 2]],
                                         bufs[j2], gsems[j2])

                    pltpu.make_async_copy(
                        h_hbm.at[sidx.at[r]], bufs[j], gsems[j]).wait()
                    pltpu.async_copy(bufs[j], acc.at[didx.at[r]],
                                     ssems[j], add=True)
                return cc

            lax.fori_loop(0, nrows // nbuf, w, 0)

            def tail_full():
                for rr in range(EROWS_PT - 4, EROWS_PT):
                    pltpu.make_async_copy(
                        bufs[rr % nbuf], acc.at[didx.at[rr]],
                        ssems[rr % nbuf]).wait()

            def tail_t15():
                base = EROWS_T15 - EROWS_T15 % nbuf
                for rr in range(base, EROWS_T15):
                    jj = rr % nbuf
                    pltpu.make_async_copy(
                        h_hbm.at[sidx.at[rr]], bufs[jj], gsems[jj]).wait()
                    pltpu.async_copy(bufs[jj], acc.at[didx.at[rr]],
                                     ssems[jj], add=True)
                for rr in range(EROWS_T15 - 4, EROWS_T15):
                    pltpu.make_async_copy(
                        bufs[rr % nbuf], acc.at[didx.at[rr]],
                        ssems[rr % nbuf]).wait()

            pl.when(tid < NS - 1)(tail_full)
            pl.when(tid == NS - 1)(tail_t15)

        run = run4 if deep else run2
        pl.when(cid == 0)(lambda: run(h_a, src1, dst1))
        pl.when(cid == 1)(lambda: run(h_b, src2, dst2))
        plsc.subcore_barrier()
        pl.when(cid == 0)(lambda: pltpu.sync_copy(acc.at[sl], out_a.at[sl]))
        pl.when(cid == 1)(lambda: pltpu.sync_copy(acc.at[sl], out_b.at[sl]))

    return agg_k


_agg128 = _make_agg(D_HID)
_agg64 = _make_agg(D_OUT)


def _norm_from_deg(dref):
    dg = dref[...]
    return jnp.where(dg > 0, lax.rsqrt(jnp.maximum(dg, 1.0)), 0.0)


def _mm_body(x_ref, w_ref, y_ref):
    y_ref[...] = jnp.dot(x_ref[...], w_ref[...],
                         preferred_element_type=jnp.float32)


def _matmul(x, w):
    n, k = x.shape
    m = w.shape[1]
    return pl.pallas_call(
        _mm_body,
        grid=(n // BLK,),
        in_specs=[pl.BlockSpec((BLK, k), lambda i: (i, 0)),
                  pl.BlockSpec((k, m), lambda i: (0, 0))],
        out_specs=pl.BlockSpec((BLK, m), lambda i: (i, 0)),
        out_shape=jax.ShapeDtypeStruct((n, m), jnp.float32),
    )(x, w)


def _prescale_body(y_ref, d0, d2, h_a, h_b):
    y = y_ref[...]
    h_a[...] = y * _norm_from_deg(d0)
    h_b[...] = y * _norm_from_deg(d2)


def _prescale(y, d0, d2):
    col = pl.BlockSpec((BLK, 1), lambda i: (i, 0))
    big = pl.BlockSpec((BLK, D_HID), lambda i: (i, 0))
    return pl.pallas_call(
        _prescale_body,
        grid=(N_NODES // BLK,),
        in_specs=[big, col, col],
        out_specs=[big, big],
        out_shape=[jax.ShapeDtypeStruct((N_PAD, D_HID), jnp.float32)] * 2,
    )(y, d0, d2)


def _tc2_body(agg_a, agg_b, d0, d1, d2, d3, attn, b1_ref, w2_ref, h_a, h_b):
    a0 = attn[0]
    a1 = attn[1]
    x1 = (a0 * (agg_a[...] * _norm_from_deg(d1))
          + a1 * (agg_b[...] * _norm_from_deg(d3))
          + (a0 + a1) * b1_ref[...])
    y2 = jnp.dot(x1, w2_ref[...], preferred_element_type=jnp.float32)
    h_a[...] = y2 * _norm_from_deg(d0)
    h_b[...] = y2 * _norm_from_deg(d2)


def _tc2(agg_a, agg_b, d0, d1, d2, d3, attn, b1, w2):
    col = pl.BlockSpec((BLK, 1), lambda i: (i, 0))
    big = pl.BlockSpec((BLK, D_HID), lambda i: (i, 0))
    return pl.pallas_call(
        _tc2_body,
        grid=(N_NODES // BLK,),
        in_specs=[
            big, big, col, col, col, col,
            pl.BlockSpec(memory_space=pltpu.SMEM),
            pl.BlockSpec((1, D_HID), lambda i: (0, 0)),
            pl.BlockSpec((D_HID, D_OUT), lambda i: (0, 0)),
        ],
        out_specs=[pl.BlockSpec((BLK, D_OUT), lambda i: (i, 0)),
                   pl.BlockSpec((BLK, D_OUT), lambda i: (i, 0))],
        out_shape=[jax.ShapeDtypeStruct((N_PAD, D_OUT), jnp.float32)] * 2,
    )(agg_a, agg_b, d0, d1, d2, d3, attn, b1, w2)


def _tc3_body(agg_a, agg_b, d1, d3, attn, b2_ref, out):
    a0 = attn[0]
    a1 = attn[1]
    out[...] = (a0 * (agg_a[...] * _norm_from_deg(d1))
                + a1 * (agg_b[...] * _norm_from_deg(d3))
                + (a0 + a1) * b2_ref[...])


def _tc3(agg_a, agg_b, d1, d3, attn, b2):
    col = pl.BlockSpec((BLK, 1), lambda i: (i, 0))
    big = pl.BlockSpec((BLK, D_OUT), lambda i: (i, 0))
    return pl.pallas_call(
        _tc3_body,
        grid=(N_NODES // BLK,),
        in_specs=[big, big, col, col,
                  pl.BlockSpec(memory_space=pltpu.SMEM),
                  pl.BlockSpec((1, D_OUT), lambda i: (0, 0))],
        out_specs=big,
        out_shape=jax.ShapeDtypeStruct((N_NODES, D_OUT), jnp.float32),
    )(agg_a, agg_b, d1, d3, attn, b2)


def kernel(features, edge_index1, edge_index2, order_attn, W1, b1, W2, b2):
    src1h = edge_index1[0].astype(jnp.int32).reshape(EROWS, LANES)
    dst1h = edge_index1[1].astype(jnp.int32).reshape(EROWS, LANES)
    src2h = edge_index2[0].astype(jnp.int32).reshape(EROWS, LANES)
    dst2h = edge_index2[1].astype(jnp.int32).reshape(EROWS, LANES)

    d0, d1, d2, d3 = _hist_k(src1h, dst1h, src2h, dst2h)
    y1 = _matmul(features, W1)
    d0 = d0.reshape(N_PAD, 1)
    d1 = d1.reshape(N_PAD, 1)
    d2 = d2.reshape(N_PAD, 1)
    d3 = d3.reshape(N_PAD, 1)

    h_a, h_b = _prescale(y1, d0, d2)

    agg1_a, agg1_b = _agg128(h_a, h_b, src1h, dst1h, src2h, dst2h)

    h2_a, h2_b = _tc2(agg1_a, agg1_b, d0, d1, d2, d3, order_attn,
                      b1.reshape(1, D_HID), W2)

    agg2_a, agg2_b = _agg64(h2_a, h2_b, src1h, dst1h, src2h, dst2h)

    return _tc3(agg2_a, agg2_b, d1, d3, order_attn, b2.reshape(1, D_OUT))


# speedup vs baseline: 1.0216x; 1.0216x over previous
import functools

import jax
import jax.numpy as jnp
from jax import lax
from jax.experimental import pallas as pl
from jax.experimental.pallas import tpu as pltpu
from jax.experimental.pallas import tpu_sc as plsc

N_NODES = 10000
D_IN = 128
D_HID = 128
D_OUT = 64
N_EDGES = 160000

NC = 2
NS = 16
LANES = 128

N_PAD = 10240
ROWS_PT = N_PAD // NS
E_PAD = 163840
EROWS = E_PAD // LANES
EROWS_PT = EROWS // NS
ICHUNK = 40
ZROWS = 32
HBATCH = 8
BLK = 2000

_sc_mesh = functools.partial(
    plsc.VectorSubcoreMesh, core_axis_name="c", subcore_axis_name="s")


@functools.partial(
    pl.kernel,
    out_type=[jax.ShapeDtypeStruct((N_PAD,), jnp.float32) for _ in range(4)],
    mesh=_sc_mesh(),
    scratch_types=[
        pltpu.VMEM((EROWS_PT, LANES), jnp.int32),
        pltpu.VMEM((EROWS_PT, LANES), jnp.int32),
        pltpu.VMEM((LANES,), jnp.float32),
        pltpu.VMEM((ROWS_PT,), jnp.float32),
        pltpu.VMEM_SHARED((N_PAD,), jnp.float32),
        pltpu.VMEM_SHARED((N_PAD,), jnp.float32),
        pltpu.SemaphoreType.DMA,
    ],
)
def _hist_k(src1, dst1, src2, dst2, d0, d1, d2, d3,
            sidx, didx, ones_v, zbuf, acc_a, acc_b, sem):
    cid = lax.axis_index("c")
    tid = lax.axis_index("s")
    for i in range(LANES // 16):
        ones_v[pl.ds(i * 16, 16)] = jnp.ones((16,), jnp.float32)

    def zb(i, c):
        zbuf[pl.ds(i * 16, 16)] = jnp.zeros((16,), jnp.float32)
        return c

    lax.fori_loop(0, ROWS_PT // 16, zb, 0)
    sl = pl.ds(tid * ROWS_PT, ROWS_PT)
    pltpu.async_copy(zbuf, acc_a.at[sl], sem)
    pltpu.async_copy(zbuf, acc_b.at[sl], sem)
    pltpu.make_async_copy(zbuf, acc_a.at[sl], sem).wait()
    pltpu.make_async_copy(zbuf, acc_b.at[sl], sem).wait()
    plsc.subcore_barrier()

    def run(src_hbm, dst_hbm):
        esl = pl.ds(tid * EROWS_PT, EROWS_PT)
        pltpu.sync_copy(src_hbm.at[esl], sidx)
        pltpu.sync_copy(dst_hbm.at[esl], didx)

        def w(i, c):
            r = i * HBATCH
            for j in range(HBATCH):
                pltpu.async_copy(ones_v, acc_a.at[sidx.at[r + j]], sem,
                                 add=True)
                pltpu.async_copy(ones_v, acc_b.at[didx.at[r + j]], sem,
                                 add=True)
            for j in range(HBATCH):
                pltpu.make_async_copy(ones_v, acc_a.at[sidx.at[r + j]],
                                      sem).wait()
                pltpu.make_async_copy(ones_v, acc_b.at[didx.at[r + j]],
                                      sem).wait()
            return c

        lax.fori_loop(0, EROWS_PT // HBATCH, w, 0)

    pl.when(cid == 0)(lambda: run(src1, dst1))
    pl.when(cid == 1)(lambda: run(src2, dst2))
    plsc.subcore_barrier()

    def out0():
        pltpu.sync_copy(acc_a.at[sl], d0.at[sl])
        pltpu.sync_copy(acc_b.at[sl], d1.at[sl])

    def out1():
        pltpu.sync_copy(acc_a.at[sl], d2.at[sl])
        pltpu.sync_copy(acc_b.at[sl], d3.at[sl])

    pl.when(cid == 0)(out0)
    pl.when(cid == 1)(out1)


def _make_agg(d):
    deep = d == D_OUT
    nbuf = 4 if deep else 2
    irows = EROWS_PT if deep else ICHUNK
    scratch = [
        pltpu.VMEM((irows, LANES), jnp.int32),
        pltpu.VMEM((irows, LANES), jnp.int32),
    ]
    scratch += [pltpu.VMEM((LANES, d), jnp.float32)] * nbuf
    scratch += [pltpu.VMEM((ZROWS, d), jnp.float32)]
    scratch += [pltpu.VMEM_SHARED((N_PAD, d), jnp.float32)]
    scratch += [pltpu.SemaphoreType.DMA] * (2 * nbuf if deep else nbuf)

    @functools.partial(
        pl.kernel,
        out_type=[jax.ShapeDtypeStruct((N_PAD, d), jnp.float32),
                  jax.ShapeDtypeStruct((N_PAD, d), jnp.float32)],
        mesh=_sc_mesh(),
        scratch_types=scratch,
        compiler_params=pltpu.CompilerParams(use_tc_tiling_on_sc=False),
    )
    def agg_k(h_a, h_b, src1, dst1, src2, dst2, out_a, out_b,
              sidx, didx, *rest):
        bufs = rest[:nbuf]
        zbuf = rest[nbuf]
        acc = rest[nbuf + 1]
        sems = rest[nbuf + 2:]
        cid = lax.axis_index("c")
        tid = lax.axis_index("s")
        sl = pl.ds(tid * ROWS_PT, ROWS_PT)

        def zb(r, c):
            for j in range(d // 16):
                zbuf[r, pl.ds(j * 16, 16)] = jnp.zeros((16,), jnp.float32)
            return c

        lax.fori_loop(0, ZROWS, zb, 0)

        def zcp(j, c):
            pltpu.async_copy(
                zbuf, acc.at[pl.ds(tid * ROWS_PT + j * ZROWS, ZROWS)],
                sems[0])
            return c

        lax.fori_loop(0, ROWS_PT // ZROWS, zcp, 0)

        def zwait(j, c):
            pltpu.make_async_copy(
                zbuf, acc.at[pl.ds(tid * ROWS_PT + j * ZROWS, ZROWS)],
                sems[0]).wait()
            return c

        lax.fori_loop(0, ROWS_PT // ZROWS, zwait, 0)
        plsc.subcore_barrier()

        def run2(h_hbm, src_hbm, dst_hbm):
            buf0, buf1 = bufs
            sem0, sem1 = sems

            def chunk(off, sz):
                base = tid * EROWS_PT + off
                pltpu.sync_copy(src_hbm.at[pl.ds(base, sz)], sidx)
                pltpu.sync_copy(dst_hbm.at[pl.ds(base, sz)], didx)
                pltpu.async_copy(h_hbm.at[sidx.at[0]], buf0, sem0)

                def w(i, cc):
                    r = 2 * i
                    pltpu.async_copy(h_hbm.at[sidx.at[r + 1]], buf1, sem1)
                    pltpu.make_async_copy(
                        h_hbm.at[sidx.at[r]], buf0, sem0).wait()
                    pltpu.sync_copy(buf0, acc.at[didx.at[r]], add=True)

                    @pl.when(r + 2 < sz)
                    def _():
                        pltpu.async_copy(h_hbm.at[sidx.at[r + 2]], buf0, sem0)

                    pltpu.make_async_copy(
                        h_hbm.at[sidx.at[r + 1]], buf1, sem1).wait()
                    pltpu.sync_copy(buf1, acc.at[didx.at[r + 1]], add=True)
                    return cc

                lax.fori_loop(0, sz // 2, w, 0)

            chunk(0, ICHUNK)
            chunk(ICHUNK, ICHUNK)

        def run4(h_hbm, src_hbm, dst_hbm):
            gsems = sems[:nbuf]
            ssems = sems[nbuf:]

            esl = pl.ds(tid * EROWS_PT, EROWS_PT)
            pltpu.sync_copy(src_hbm.at[esl], sidx)
            pltpu.sync_copy(dst_hbm.at[esl], didx)
            pltpu.async_copy(h_hbm.at[sidx.at[0]], bufs[0], gsems[0])
            pltpu.async_copy(h_hbm.at[sidx.at[1]], bufs[1], gsems[1])

            def w(i, cc):
                for j in range(nbuf):
                    r = nbuf * i + j
                    j2 = (j + 2) % nbuf

                    @pl.when(r + 2 < EROWS_PT)
                    def _():
                        @pl.when(r >= 2)
                        def _():
                            pltpu.make_async_copy(
                                bufs[j2], acc.at[didx.at[r - 2]],
                                ssems[j2]).wait()

                        pltpu.async_copy(h_hbm.at[sidx.at[r + 2]],
                                         bufs[j2], gsems[j2])

                    pltpu.make_async_copy(
                        h_hbm.at[sidx.at[r]], bufs[j], gsems[j]).wait()
                    pltpu.async_copy(bufs[j], acc.at[didx.at[r]],
                                     ssems[j], add=True)
                return cc

            lax.fori_loop(0, EROWS_PT // nbuf, w, 0)
            for rr in range(EROWS_PT - 4, EROWS_PT):
                jj = rr % nbuf
                pltpu.make_async_copy(
                    bufs[jj], acc.at[didx.at[rr]], ssems[jj]).wait()

        run = run4 if deep else run2
        pl.when(cid == 0)(lambda: run(h_a, src1, dst1))
        pl.when(cid == 1)(lambda: run(h_b, src2, dst2))
        plsc.subcore_barrier()
        pl.when(cid == 0)(lambda: pltpu.sync_copy(acc.at[sl], out_a.at[sl]))
        pl.when(cid == 1)(lambda: pltpu.sync_copy(acc.at[sl], out_b.at[sl]))

    return agg_k


_agg128 = _make_agg(D_HID)
_agg64 = _make_agg(D_OUT)


def _norm_from_deg(dref):
    dg = dref[...]
    return jnp.where(dg > 0, lax.rsqrt(jnp.maximum(dg, 1.0)), 0.0)


def _mm_body(x_ref, w_ref, y_ref):
    y_ref[...] = jnp.dot(x_ref[...], w_ref[...],
                         preferred_element_type=jnp.float32)


def _matmul(x, w):
    n, k = x.shape
    m = w.shape[1]
    return pl.pallas_call(
        _mm_body,
        grid=(n // BLK,),
        in_specs=[pl.BlockSpec((BLK, k), lambda i: (i, 0)),
                  pl.BlockSpec((k, m), lambda i: (0, 0))],
        out_specs=pl.BlockSpec((BLK, m), lambda i: (i, 0)),
        out_shape=jax.ShapeDtypeStruct((n, m), jnp.float32),
    )(x, w)


def _prescale_body(y_ref, d0, d2, h_a, h_b):
    y = y_ref[...]
    h_a[...] = y * _norm_from_deg(d0)
    h_b[...] = y * _norm_from_deg(d2)


def _prescale(y, d0, d2):
    col = pl.BlockSpec((BLK, 1), lambda i: (i, 0))
    big = pl.BlockSpec((BLK, D_HID), lambda i: (i, 0))
    return pl.pallas_call(
        _prescale_body,
        grid=(N_NODES // BLK,),
        in_specs=[big, col, col],
        out_specs=[big, big],
        out_shape=[jax.ShapeDtypeStruct((N_PAD, D_HID), jnp.float32)] * 2,
    )(y, d0, d2)


def _tc2_body(agg_a, agg_b, d0, d1, d2, d3, attn, b1_ref, w2_ref, h_a, h_b):
    a0 = attn[0]
    a1 = attn[1]
    x1 = (a0 * (agg_a[...] * _norm_from_deg(d1))
          + a1 * (agg_b[...] * _norm_from_deg(d3))
          + (a0 + a1) * b1_ref[...])
    y2 = jnp.dot(x1, w2_ref[...], preferred_element_type=jnp.float32)
    h_a[...] = y2 * _norm_from_deg(d0)
    h_b[...] = y2 * _norm_from_deg(d2)


def _tc2(agg_a, agg_b, d0, d1, d2, d3, attn, b1, w2):
    col = pl.BlockSpec((BLK, 1), lambda i: (i, 0))
    big = pl.BlockSpec((BLK, D_HID), lambda i: (i, 0))
    return pl.pallas_call(
        _tc2_body,
        grid=(N_NODES // BLK,),
        in_specs=[
            big, big, col, col, col, col,
            pl.BlockSpec(memory_space=pltpu.SMEM),
            pl.BlockSpec((1, D_HID), lambda i: (0, 0)),
            pl.BlockSpec((D_HID, D_OUT), lambda i: (0, 0)),
        ],
        out_specs=[pl.BlockSpec((BLK, D_OUT), lambda i: (i, 0)),
                   pl.BlockSpec((BLK, D_OUT), lambda i: (i, 0))],
        out_shape=[jax.ShapeDtypeStruct((N_PAD, D_OUT), jnp.float32)] * 2,
    )(agg_a, agg_b, d0, d1, d2, d3, attn, b1, w2)


def _tc3_body(agg_a, agg_b, d1, d3, attn, b2_ref, out):
    a0 = attn[0]
    a1 = attn[1]
    out[...] = (a0 * (agg_a[...] * _norm_from_deg(d1))
                + a1 * (agg_b[...] * _norm_from_deg(d3))
                + (a0 + a1) * b2_ref[...])


def _tc3(agg_a, agg_b, d1, d3, attn, b2):
    col = pl.BlockSpec((BLK, 1), lambda i: (i, 0))
    big = pl.BlockSpec((BLK, D_OUT), lambda i: (i, 0))
    return pl.pallas_call(
        _tc3_body,
        grid=(N_NODES // BLK,),
        in_specs=[big, big, col, col,
                  pl.BlockSpec(memory_space=pltpu.SMEM),
                  pl.BlockSpec((1, D_OUT), lambda i: (0, 0))],
        out_specs=big,
        out_shape=jax.ShapeDtypeStruct((N_NODES, D_OUT), jnp.float32),
    )(agg_a, agg_b, d1, d3, attn, b2)


def _pad_edges(idx, fill):
    return jnp.concatenate(
        [idx.astype(jnp.int32), fill]).reshape(EROWS, LANES)


def kernel(features, edge_index1, edge_index2, order_attn, W1, b1, W2, b2):
    npad = E_PAD - N_EDGES
    pad = N_NODES + (jnp.arange(npad, dtype=jnp.int32) % (N_PAD - N_NODES))

    src1h = _pad_edges(edge_index1[0], pad)
    dst1h = _pad_edges(edge_index1[1], pad)
    src2h = _pad_edges(edge_index2[0], pad)
    dst2h = _pad_edges(edge_index2[1], pad)

    d0, d1, d2, d3 = _hist_k(src1h, dst1h, src2h, dst2h)
    y1 = _matmul(features, W1)
    d0 = d0.reshape(N_PAD, 1)
    d1 = d1.reshape(N_PAD, 1)
    d2 = d2.reshape(N_PAD, 1)
    d3 = d3.reshape(N_PAD, 1)

    h_a, h_b = _prescale(y1, d0, d2)

    agg1_a, agg1_b = _agg128(h_a, h_b, src1h, dst1h, src2h, dst2h)

    h2_a, h2_b = _tc2(agg1_a, agg1_b, d0, d1, d2, d3, order_attn,
                      b1.reshape(1, D_HID), W2)

    agg2_a, agg2_b = _agg64(h2_a, h2_b, src1h, dst1h, src2h, dst2h)

    return _tc3(agg2_a, agg2_b, d1, d3, order_attn, b2.reshape(1, D_OUT))
